# baseline (speedup 1.0000x reference)
import functools

import jax
import jax.numpy as jnp
from jax import lax
from jax.experimental import pallas as pl
from jax.experimental.pallas import tpu as pltpu
from jax.experimental.pallas import tpu_sc as plsc

_LAMBDA = 0.001
_LN_EPS = 1e-5
_POOL_EPS = 1e-8

_NC = 2
_NS = 16
_NW = _NC * _NS
_LANE = 16


def _sc_pool(hist2, hpad, tspad, gidx, uid, rvals, avals, rbounds, abounds,
             user_table, item128, genre128, bucket128):
  B = hist2.shape[0]
  LH = hist2.shape[2]
  L = hist2.shape[1] * LH
  LP = hpad.shape[1]
  GP = gidx.shape[1]
  D = user_table.shape[1]
  DG = D // _LANE
  NB = rbounds.shape[0]
  n_bounds = 19
  BW = B // _NW
  CB = 16
  NCH = BW // CB

  mesh = plsc.VectorSubcoreMesh(core_axis_name="c", subcore_axis_name="s",
                                num_cores=_NC, num_subcores=_NS)

  @functools.partial(
      pl.kernel,
      out_type=jax.ShapeDtypeStruct((B, 4 * D + 2 * _LANE), jnp.float32),
      mesh=mesh,
      scratch_types=dict(
          uid_v=pltpu.VMEM((BW,), jnp.int32),
          urows_v=pltpu.VMEM((BW, D), jnp.float32),
          rv_v=pltpu.VMEM((BW,), jnp.float32),
          av_v=pltpu.VMEM((BW,), jnp.float32),
          ridx_v=pltpu.VMEM((BW,), jnp.int32),
          aidx_v=pltpu.VMEM((BW,), jnp.int32),
          crows_v=pltpu.VMEM((BW, 2 * D), jnp.float32),
          arows_v=pltpu.VMEM((BW, 2 * D), jnp.float32),
          rb_v=pltpu.VMEM((NB,), jnp.float32),
          ab_v=pltpu.VMEM((NB,), jnp.float32),
          hidx_v=pltpu.VMEM((CB, 2, LH), jnp.int32),
          hpad_v=pltpu.VMEM((CB, LP), jnp.int32),
          tspad_v=pltpu.VMEM((CB, LP), jnp.float32),
          gidx_v=pltpu.VMEM((CB, GP), jnp.int32),
          rows_v=pltpu.VMEM((L, 2 * D), jnp.float32),
          grows_v=pltpu.VMEM((GP, 2 * D), jnp.float32),
          xo_v=pltpu.VMEM((CB, 4 * D + 2 * _LANE), jnp.float32),
          sem=pltpu.SemaphoreType.DMA,
          usem=pltpu.SemaphoreType.DMA,
      ),
  )
  def sc_kernel(hist2_h, hpad_h, tspad_h, gidx_h, uid_h, rv_h, av_h, rb_h,
                ab_h, utab_h, itab_h, gtab_h, btab_h, out_h,
                uid_v, urows_v, rv_v, av_v, ridx_v, aidx_v, crows_v, arows_v,
                rb_v, ab_v, hidx_v, hpad_v, tspad_v, gidx_v, rows_v, grows_v,
                xo_v, sem, usem):
    wid = lax.axis_index("s") * _NC + lax.axis_index("c")
    base = wid * BW

    pltpu.sync_copy(uid_h.at[pl.ds(base, BW)], uid_v)
    pltpu.sync_copy(rv_h.at[pl.ds(base, BW)], rv_v)
    pltpu.sync_copy(av_h.at[pl.ds(base, BW)], av_v)
    pltpu.sync_copy(rb_h.at[:], rb_v)
    pltpu.sync_copy(ab_h.at[:], ab_v)

    def ugather_body(g, _):
      uid16 = uid_v[pl.ds(g * _LANE, _LANE)]
      descs = []
      for k in range(_LANE):
        descs.append(pltpu.async_copy(utab_h.at[uid16[k]],
                                      urows_v.at[g * _LANE + k], usem))
      for c in descs:
        c.wait()
      return 0

    lax.fori_loop(0, BW // _LANE, ugather_body, 0)

    rbA = rb_v[pl.ds(0, _LANE)]
    rbB = rb_v[pl.ds(_LANE, _LANE)]
    abA = ab_v[pl.ds(0, _LANE)]
    abB = ab_v[pl.ds(_LANE, _LANE)]

    def bucket_body(i, _):
      rv16 = rv_v[pl.ds(i * _LANE, _LANE)]
      av16 = av_v[pl.ds(i * _LANE, _LANE)]
      ri = jnp.zeros((_LANE,), jnp.float32)
      ai = jnp.zeros((_LANE,), jnp.float32)
      for j in range(n_bounds):
        rbj = rbA[j] if j < _LANE else rbB[j - _LANE]
        abj = abA[j] if j < _LANE else abB[j - _LANE]
        ri = ri + jnp.maximum(jnp.sign(rv16 - rbj), 0.0)
        ai = ai + jnp.maximum(jnp.sign(av16 - abj), 0.0)
      ridx_v[pl.ds(i * _LANE, _LANE)] = ri.astype(jnp.int32)
      aidx_v[pl.ds(i * _LANE, _LANE)] = ai.astype(jnp.int32)
      return 0

    lax.fori_loop(0, BW // _LANE, bucket_body, 0)

    pltpu.async_copy(btab_h.at[ridx_v], crows_v, sem).wait()
    pltpu.async_copy(btab_h.at[aidx_v], arows_v, sem).wait()

    def chunk_body(g, _):
      cb = base + g * CB
      pltpu.sync_copy(hist2_h.at[pl.ds(cb, CB)], hidx_v)
      pltpu.sync_copy(hpad_h.at[pl.ds(cb, CB)], hpad_v)
      pltpu.sync_copy(tspad_h.at[pl.ds(cb, CB)], tspad_v)
      pltpu.sync_copy(gidx_h.at[pl.ds(cb, CB)], gidx_v)

      def row_body(b, _):
        r = g * CB + b
        c1 = pltpu.async_copy(itab_h.at[hidx_v.at[b, 0]],
                              rows_v.at[pl.ds(0, LH)], sem)
        c2 = pltpu.async_copy(itab_h.at[hidx_v.at[b, 1]],
                              rows_v.at[pl.ds(LH, LH)], sem)
        c3 = pltpu.async_copy(gtab_h.at[gidx_v.at[b]], grows_v, sem)

        wlo, whi = [], []
        den = jnp.zeros((_LANE,), jnp.float32)
        for t in range(LP // _LANE):
          ts16 = tspad_v[b, pl.ds(t * _LANE, _LANE)]
          hi16 = hpad_v[b, pl.ds(t * _LANE, _LANE)]
          w = jnp.exp(ts16 * (-_LAMBDA)) * jnp.minimum(hi16, 1).astype(
              jnp.float32)
          par = (hi16 & 1).astype(jnp.float32)
          wh = w * par
          whi.append(wh)
          wlo.append(w - wh)
          den = den + w

        gm = jnp.minimum(gidx_v[b, :], 1).astype(jnp.float32)

        c1.wait()
        c2.wait()
        c3.wait()

        acc = [jnp.zeros((_LANE,), jnp.float32) for _ in range(DG)]
        for l in range(L):
          wl = wlo[l // _LANE][l % _LANE]
          wh = whi[l // _LANE][l % _LANE]
          for d in range(DG):
            acc[d] = (acc[d] + wl * rows_v[l, pl.ds(d * _LANE, _LANE)]
                      + wh * rows_v[l, pl.ds(D + d * _LANE, _LANE)])

        gacc = [jnp.zeros((_LANE,), jnp.float32) for _ in range(DG)]
        for j in range(GP):
          m_b = gm[j]
          for d in range(DG):
            gacc[d] = gacc[d] + m_b * grows_v[j, pl.ds(d * _LANE, _LANE)]

        for d in range(DG):
          sl = pl.ds(d * _LANE, _LANE)
          xo_v[b, pl.ds(d * _LANE, _LANE)] = urows_v[r, sl]
          xo_v[b, pl.ds(D + d * _LANE, _LANE)] = acc[d]
          xo_v[b, pl.ds(2 * D + d * _LANE, _LANE)] = gacc[d]
          xo_v[b, pl.ds(3 * D + d * _LANE, _LANE)] = (
              crows_v[r, sl] + arows_v[r, pl.ds(D + d * _LANE, _LANE)])
        xo_v[b, pl.ds(4 * D, _LANE)] = den
        xo_v[b, pl.ds(4 * D + _LANE, _LANE)] = gm
        return 0

      lax.fori_loop(0, CB, row_body, 0)
      pltpu.sync_copy(xo_v, out_h.at[pl.ds(cb, CB)])
      return 0

    lax.fori_loop(0, NCH, chunk_body, 0)

  return sc_kernel(hist2, hpad, tspad, gidx, uid, rvals, avals, rbounds,
                   abounds, user_table, item128, genre128, bucket128)


def _mlp_body(x_ref, w1_ref, b1_ref, g_ref, be_ref, w2_ref, b2_ref, o_ref):
  xr = x_ref[...]
  D = w2_ref.shape[0]
  den = jnp.sum(xr[:, 4 * D:4 * D + _LANE], axis=-1, keepdims=True)
  gden = jnp.sum(xr[:, 4 * D + _LANE:4 * D + 2 * _LANE], axis=-1,
                 keepdims=True)
  hinv = 1.0 / (den + _POOL_EPS)
  ginv = 1.0 / (gden + _POOL_EPS)
  x = jnp.concatenate(
      [xr[:, :D], xr[:, D:2 * D] * hinv, xr[:, 2 * D:3 * D] * ginv,
       xr[:, 3 * D:4 * D]], axis=1)
  h = lax.dot_general(x, w1_ref[...], (((1,), (1,)), ((), ())),
                      preferred_element_type=jnp.float32) + b1_ref[...]
  mu = jnp.mean(h, axis=-1, keepdims=True)
  d = h - mu
  var = jnp.mean(d * d, axis=-1, keepdims=True)
  hn = d * lax.rsqrt(var + _LN_EPS) * g_ref[...] + be_ref[...]
  hr = jnp.maximum(hn, 0.0)
  o = lax.dot_general(hr, w2_ref[...], (((1,), (1,)), ((), ())),
                      preferred_element_type=jnp.float32) + b2_ref[...]
  n2 = jnp.sum(o * o, axis=-1, keepdims=True)
  o_ref[...] = o / jnp.maximum(jnp.sqrt(n2), 1e-12)


def _mlp(x, W1, b1, ln_g, ln_b, W2, b2):
  B, F = x.shape
  H = W1.shape[0]
  D = W2.shape[0]
  BT = 512
  grid = (B // BT,)
  return pl.pallas_call(
      _mlp_body,
      grid=grid,
      in_specs=[
          pl.BlockSpec((BT, F), lambda i: (i, 0)),
          pl.BlockSpec((H, 4 * D), lambda i: (0, 0)),
          pl.BlockSpec((1, H), lambda i: (0, 0)),
          pl.BlockSpec((1, H), lambda i: (0, 0)),
          pl.BlockSpec((1, H), lambda i: (0, 0)),
          pl.BlockSpec((D, H), lambda i: (0, 0)),
          pl.BlockSpec((1, D), lambda i: (0, 0)),
      ],
      out_specs=pl.BlockSpec((BT, D), lambda i: (i, 0)),
      out_shape=jax.ShapeDtypeStruct((B, D), jnp.float32),
  )(x, W1, b1, ln_g, ln_b, W2, b2)


def kernel(user_id, history, history_ts_diff, top_genres, avg_rating, activity,
           user_table, item_table, genre_table, rating_table, activity_table,
           avg_rating_bounds, activity_bounds, W1, b1, ln_g, ln_b, W2, b2):
  B, L = history.shape
  D = item_table.shape[1]
  hist_i = history.astype(jnp.int32)
  item128 = item_table.reshape(item_table.shape[0] // 2, 2 * D)
  hist2 = (hist_i // 2).reshape(B, 2, L // 2)
  genre128 = jnp.pad(genre_table, ((0, 0), (0, D)))
  bucket128 = jnp.concatenate([rating_table, activity_table], axis=1)

  LP = ((L + _LANE - 1) // _LANE) * _LANE
  hpad = jnp.pad(hist_i, ((0, 0), (0, LP - L)))
  tspad = jnp.pad(history_ts_diff.astype(jnp.float32), ((0, 0), (0, LP - L)))
  G = top_genres.shape[1]
  gidx = jnp.pad(top_genres.astype(jnp.int32), ((0, 0), (0, _LANE - G)))
  uid = user_id.astype(jnp.int32)
  fmax = jnp.finfo(jnp.float32).max
  nb = avg_rating_bounds.shape[0]
  rb = jnp.pad(avg_rating_bounds.astype(jnp.float32), (0, 32 - nb),
               constant_values=fmax)
  ab = jnp.pad(activity_bounds.astype(jnp.float32), (0, 32 - nb),
               constant_values=fmax)

  x = _sc_pool(hist2, hpad, tspad, gidx, uid,
               avg_rating.astype(jnp.float32), activity.astype(jnp.float32),
               rb, ab, user_table, item128, genre128, bucket128)

  return _mlp(x, W1, b1.reshape(1, -1), ln_g.reshape(1, -1),
              ln_b.reshape(1, -1), W2, b2.reshape(1, -1))
